# Initial kernel scaffold; baseline (speedup 1.0000x reference)
import functools

import jax
import jax.numpy as jnp
from jax import lax
from jax.experimental import pallas as pl
from jax.experimental.pallas import tpu as pltpu
from jax.experimental.pallas import tpu_sc as plsc

N = 10000
D = 128
NC, NS, L = 2, 16, 16
BM = 1000
GRID_M = N // BM
CH = 128
NCH = 79
NG = NC * NS
E_PAD = NG * NCH * CH
ROWS_PER_TILE = N // NS


def _mm0_body(x_ref, w_ref, p_ref):
    p = jnp.dot(x_ref[...], w_ref[...], preferred_element_type=jnp.float32)
    p_ref[0] = p[:, :D]
    p_ref[1] = p[:, D:]


def _mm0(x, W0):
    return pl.pallas_call(
        _mm0_body,
        grid=(GRID_M,),
        in_specs=[pl.BlockSpec((BM, 128), lambda i: (i, 0)),
                  pl.BlockSpec((128, 256), lambda i: (0, 0))],
        out_specs=pl.BlockSpec((2, BM, D), lambda i: (0, i, 0)),
        out_shape=jax.ShapeDtypeStruct((2, N, D), jnp.float32),
    )(x, W0)


def _act_mm_body(has_prev, emit_h, g_ref, b_ref, *rest):
    if has_prev:
        prev_ref, w_ref = rest[0], rest[1]
        outs = rest[2:]
    else:
        w_ref = rest[0]
        outs = rest[1:]
    h0 = jax.nn.relu(g_ref[0] + b_ref[0])
    h1 = jax.nn.relu(g_ref[1] + b_ref[1])
    if has_prev:
        h0 = h0 + prev_ref[0]
        h1 = h1 + prev_ref[1]
    p = (jnp.dot(h0, w_ref[0], preferred_element_type=jnp.float32)
         + jnp.dot(h1, w_ref[1], preferred_element_type=jnp.float32))
    if emit_h:
        h_ref, p_ref = outs
        h_ref[0] = h0
        h_ref[1] = h1
        p_ref[0] = p[:, :D]
        p_ref[1] = p[:, D:]
    else:
        (p_ref,) = outs
        p_ref[...] = p


def _act_mm(g, b2, prev, Wr, emit_h):
    dout = Wr.shape[2]
    has_prev = prev is not None
    in_specs = [pl.BlockSpec((2, BM, D), lambda i: (0, i, 0)),
                pl.BlockSpec((2, 1, D), lambda i: (0, 0, 0))]
    args = [g, b2]
    if has_prev:
        in_specs.append(pl.BlockSpec((2, BM, D), lambda i: (0, i, 0)))
        args.append(prev)
    in_specs.append(pl.BlockSpec((2, 128, dout), lambda i: (0, 0, 0)))
    args.append(Wr)
    if emit_h:
        out_specs = [pl.BlockSpec((2, BM, D), lambda i: (0, i, 0)),
                     pl.BlockSpec((2, BM, D), lambda i: (0, i, 0))]
        out_shape = [jax.ShapeDtypeStruct((2, N, D), jnp.float32),
                     jax.ShapeDtypeStruct((2, N, D), jnp.float32)]
    else:
        out_specs = pl.BlockSpec((BM, dout), lambda i: (i, 0))
        out_shape = jax.ShapeDtypeStruct((N, dout), jnp.float32)
    return pl.pallas_call(
        functools.partial(_act_mm_body, has_prev, emit_h),
        grid=(GRID_M,),
        in_specs=in_specs,
        out_specs=out_specs,
        out_shape=out_shape,
    )(*args)


def _final_body(parts_ref, b_ref, out_ref):
    out_ref[...] = parts_ref[0] + parts_ref[1] + b_ref[...]


def _final(parts, b3):
    return pl.pallas_call(
        _final_body,
        grid=(GRID_M,),
        in_specs=[pl.BlockSpec((2, BM, D), lambda i: (0, i, 0)),
                  pl.BlockSpec((1, D), lambda i: (0, 0))],
        out_specs=pl.BlockSpec((BM, D), lambda i: (i, 0)),
        out_shape=jax.ShapeDtypeStruct((N, D), jnp.float32),
    )(parts, b3.reshape(1, D))


def _spmm_body(col_split, p_hbm, src_hbm, dst_hbm, w_hbm, out_hbm,
               acc, srcb, dstb, wb, rows, gsem):
    c = lax.axis_index("c")
    s = lax.axis_index("s")
    zero16 = jnp.zeros((16,), jnp.float32)

    def zrow(r, _):
        rr = rows.at[r]
        for q in range(8):
            rr[pl.ds(q * 16, 16)] = zero16
        return 0
    lax.fori_loop(0, CH, zrow, 0)
    base = s * ROWS_PER_TILE
    for k in range(5):
        sz = CH if k < 4 else ROWS_PER_TILE - 4 * CH
        pltpu.sync_copy(rows.at[pl.ds(0, sz)],
                        acc.at[pl.ds(base + k * CH, sz)])
    plsc.subcore_barrier()

    if col_split:
        groups = (2 * s, 2 * s + 1)
        table = p_hbm.at[c]
    else:
        groups = (c * NS + s,)
        table = p_hbm.at[0]

    for g in groups:
        pltpu.sync_copy(src_hbm.at[g], srcb)
        pltpu.sync_copy(dst_hbm.at[g], dstb)
        pltpu.sync_copy(w_hbm.at[g], wb)

        def chunk_body(j, _):
            pltpu.async_copy(table.at[srcb.at[j]], rows, gsem).wait()

            def row_body(r, _):
                wv = plsc.load_gather(
                    wb, [jnp.full((16,), j, jnp.int32),
                         jnp.full((16,), r, jnp.int32)])
                rr = rows.at[r]
                for q in range(8):
                    sl = pl.ds(q * 16, 16)
                    rr[sl] = rr[sl] * wv
                return 0
            lax.fori_loop(0, CH, row_body, 0)
            pltpu.sync_copy(rows, acc.at[dstb.at[j]], add=True)
            return 0
        lax.fori_loop(0, NCH, chunk_body, 0)

    plsc.subcore_barrier()
    pltpu.sync_copy(acc.at[pl.ds(base, ROWS_PER_TILE)],
                    out_hbm.at[c, pl.ds(base, ROWS_PER_TILE)])


def _make_spmm(col_split):
    mesh = plsc.VectorSubcoreMesh(core_axis_name="c", subcore_axis_name="s",
                                  num_cores=NC, num_subcores=NS)
    return pl.kernel(
        functools.partial(_spmm_body, col_split),
        out_type=jax.ShapeDtypeStruct((2, N, D), jnp.float32),
        mesh=mesh,
        scratch_types=[
            pltpu.VMEM_SHARED((N, D), jnp.float32),
            pltpu.VMEM((NCH, CH), jnp.int32),
            pltpu.VMEM((NCH, CH), jnp.int32),
            pltpu.VMEM((NCH, CH), jnp.float32),
            pltpu.VMEM((CH, D), jnp.float32),
            pltpu.SemaphoreType.DMA,
        ],
    )


def kernel(x, edge_index, edge_weight, W0, b0, W1, b1, W2, b2, W3, b3):
    E = edge_index.shape[1]
    pad = E_PAD - E
    src3 = jnp.pad(edge_index[0], (0, pad)).reshape(NG, NCH, CH)
    dst3 = jnp.pad(edge_index[1], (0, pad)).reshape(NG, NCH, CH)
    w3 = jnp.pad(edge_weight, (0, pad)).reshape(NG, NCH, CH)

    spmm256 = _make_spmm(True)
    spmm128 = _make_spmm(False)

    p = _mm0(x, W0)
    g0 = spmm256(p, src3, dst3, w3)
    h0, p1 = _act_mm(g0, b0.reshape(2, 1, D), None,
                     W1.reshape(2, 128, 256), True)
    g1 = spmm256(p1, src3, dst3, w3)
    h1, p2 = _act_mm(g1, b1.reshape(2, 1, D), h0,
                     W2.reshape(2, 128, 256), True)
    g2 = spmm256(p2, src3, dst3, w3)
    p3 = _act_mm(g2, b2.reshape(2, 1, D), h1,
                 W3.reshape(2, 128, 128), False)
    parts = spmm128(p3.reshape(1, N, D), src3, dst3, w3)
    return _final(parts, b3)


# speedup vs baseline: 3.6517x; 3.6517x over previous
import functools

import jax
import jax.numpy as jnp
from jax import lax
from jax.experimental import pallas as pl
from jax.experimental.pallas import tpu as pltpu
from jax.experimental.pallas import tpu_sc as plsc

N = 10000
N_PAD = 10240
D = 128
NC, NS, L = 2, 16, 16
BM = 1000
GRID_M = N // BM
CH = 128
NCH = 79
NG = NC * NS
E_PAD = NG * NCH * CH
ROWS_PER_TILE = N_PAD // NS


def _mm0_body(x_ref, w_ref, p_ref):
    p = jnp.dot(x_ref[...], w_ref[...], preferred_element_type=jnp.float32)
    p_ref[0] = p[:, :D]
    p_ref[1] = p[:, D:]


def _mm0(x, W0):
    return pl.pallas_call(
        _mm0_body,
        grid=(GRID_M,),
        in_specs=[pl.BlockSpec((BM, 128), lambda i: (i, 0)),
                  pl.BlockSpec((128, 256), lambda i: (0, 0))],
        out_specs=pl.BlockSpec((2, BM, D), lambda i: (0, i, 0)),
        out_shape=jax.ShapeDtypeStruct((2, N, D), jnp.float32),
    )(x, W0)


def _act_mm_body(has_prev, emit_h, g_ref, b_ref, *rest):
    if has_prev:
        prev_ref, w_ref = rest[0], rest[1]
        outs = rest[2:]
    else:
        w_ref = rest[0]
        outs = rest[1:]
    h0 = jax.nn.relu(g_ref[0] + b_ref[0])
    h1 = jax.nn.relu(g_ref[1] + b_ref[1])
    if has_prev:
        h0 = h0 + prev_ref[0]
        h1 = h1 + prev_ref[1]
    p = (jnp.dot(h0, w_ref[0], preferred_element_type=jnp.float32)
         + jnp.dot(h1, w_ref[1], preferred_element_type=jnp.float32))
    if emit_h:
        h_ref, p_ref = outs
        h_ref[0] = h0
        h_ref[1] = h1
        p_ref[0] = p[:, :D]
        p_ref[1] = p[:, D:]
    else:
        (p_ref,) = outs
        p_ref[...] = p


def _act_mm(g, b2, prev, Wr, emit_h):
    dout = Wr.shape[2]
    has_prev = prev is not None
    in_specs = [pl.BlockSpec((2, BM, D), lambda i: (0, i, 0)),
                pl.BlockSpec((2, 1, D), lambda i: (0, 0, 0))]
    args = [g, b2]
    if has_prev:
        in_specs.append(pl.BlockSpec((2, BM, D), lambda i: (0, i, 0)))
        args.append(prev)
    in_specs.append(pl.BlockSpec((2, 128, dout), lambda i: (0, 0, 0)))
    args.append(Wr)
    if emit_h:
        out_specs = [pl.BlockSpec((2, BM, D), lambda i: (0, i, 0)),
                     pl.BlockSpec((2, BM, D), lambda i: (0, i, 0))]
        out_shape = [jax.ShapeDtypeStruct((2, N, D), jnp.float32),
                     jax.ShapeDtypeStruct((2, N, D), jnp.float32)]
    else:
        out_specs = pl.BlockSpec((BM, dout), lambda i: (i, 0))
        out_shape = jax.ShapeDtypeStruct((N, dout), jnp.float32)
    return pl.pallas_call(
        functools.partial(_act_mm_body, has_prev, emit_h),
        grid=(GRID_M,),
        in_specs=in_specs,
        out_specs=out_specs,
        out_shape=out_shape,
    )(*args)


def _final_body(parts_ref, b_ref, out_ref):
    out_ref[...] = parts_ref[0] + parts_ref[1] + b_ref[...]


def _final(parts, b3):
    return pl.pallas_call(
        _final_body,
        grid=(GRID_M,),
        in_specs=[pl.BlockSpec((2, BM, D), lambda i: (0, i, 0)),
                  pl.BlockSpec((1, D), lambda i: (0, 0))],
        out_specs=pl.BlockSpec((BM, D), lambda i: (i, 0)),
        out_shape=jax.ShapeDtypeStruct((N, D), jnp.float32),
    )(parts, b3.reshape(1, D))


_GDN = lax.GatherDimensionNumbers(
    offset_dims=(), collapsed_slice_dims=(0,), start_index_map=(0,))


def _lane_splat(v, i):
    idx = jnp.full((L, 1), i, jnp.int32)
    return lax.gather(v, idx, _GDN, (1,),
                      mode=lax.GatherScatterMode.PROMISE_IN_BOUNDS)

def _spmm_body(col_split, p_hbm, src_hbm, dst_hbm, w_hbm, out_hbm,
               acc, srcb, dstb, wb, rows, gsem):
    c = lax.axis_index("c")
    s = lax.axis_index("s")
    zero16 = jnp.zeros((16,), jnp.float32)

    def zrow(r, _):
        rr = rows.at[r]
        for q in range(8):
            rr[pl.ds(q * 16, 16)] = zero16
        return 0
    lax.fori_loop(0, CH, zrow, 0)
    base = s * ROWS_PER_TILE
    for k in range(ROWS_PER_TILE // CH):
        pltpu.sync_copy(rows, acc.at[pl.ds(base + k * CH, CH)])
    plsc.subcore_barrier()

    if col_split:
        groups = (2 * s, 2 * s + 1)
        table = p_hbm.at[c]
    else:
        groups = (c * NS + s,)
        table = p_hbm.at[0]

    for g in groups:
        pltpu.sync_copy(src_hbm.at[g], srcb)
        pltpu.sync_copy(dst_hbm.at[g], dstb)
        pltpu.sync_copy(w_hbm.at[g], wb)

        def chunk_body(j, _):
            pltpu.async_copy(table.at[srcb.at[j]], rows, gsem).wait()
            wrow = wb.at[j]

            def tgroup(t, _):
                wv16 = wrow[pl.ds(t * L, L)]
                for r16 in range(L):
                    wsp = _lane_splat(wv16, r16)
                    rr = rows.at[t * L + r16]
                    for q in range(8):
                        sl = pl.ds(q * L, L)
                        rr[sl] = rr[sl] * wsp
                return 0
            lax.fori_loop(0, CH // L, tgroup, 0)
            pltpu.sync_copy(rows, acc.at[dstb.at[j]], add=True)
            return 0
        lax.fori_loop(0, NCH, chunk_body, 0)

    plsc.subcore_barrier()
    pltpu.sync_copy(acc.at[pl.ds(base, ROWS_PER_TILE)],
                    out_hbm.at[c, pl.ds(base, ROWS_PER_TILE)])


def _make_spmm(col_split):
    mesh = plsc.VectorSubcoreMesh(core_axis_name="c", subcore_axis_name="s",
                                  num_cores=NC, num_subcores=NS)
    return pl.kernel(
        functools.partial(_spmm_body, col_split),
        out_type=jax.ShapeDtypeStruct((2, N_PAD, D), jnp.float32),
        mesh=mesh,
        scratch_types=[
            pltpu.VMEM_SHARED((N_PAD, D), jnp.float32),
            pltpu.VMEM((NCH, CH), jnp.int32),
            pltpu.VMEM((NCH, CH), jnp.int32),
            pltpu.VMEM((NCH, CH), jnp.float32),
            pltpu.VMEM((CH, D), jnp.float32),
            pltpu.SemaphoreType.DMA,
        ],
    )


def kernel(x, edge_index, edge_weight, W0, b0, W1, b1, W2, b2, W3, b3):
    E = edge_index.shape[1]
    pad = E_PAD - E
    src3 = jnp.pad(edge_index[0], (0, pad)).reshape(NG, NCH, CH)
    dst3 = jnp.pad(edge_index[1], (0, pad)).reshape(NG, NCH, CH)
    w3 = jnp.pad(edge_weight, (0, pad)).reshape(NG, NCH, CH)

    spmm256 = _make_spmm(True)
    spmm128 = _make_spmm(False)

    p = _mm0(x, W0)
    g0 = spmm256(p, src3, dst3, w3)
    h0, p1 = _act_mm(g0, b0.reshape(2, 1, D), None,
                     W1.reshape(2, 128, 256), True)
    g1 = spmm256(p1, src3, dst3, w3)
    h1, p2 = _act_mm(g1, b1.reshape(2, 1, D), h0,
                     W2.reshape(2, 128, 256), True)
    g2 = spmm256(p2, src3, dst3, w3)
    p3 = _act_mm(g2, b2.reshape(2, 1, D), h1,
                 W3.reshape(2, 128, 128), False)
    parts = spmm128(p3.reshape(1, N, D), src3, dst3, w3)
    return _final(parts, b3)
